# baseline (speedup 1.0000x reference)
import functools

import jax
import jax.numpy as jnp
from jax import lax
from jax.experimental import pallas as pl
from jax.experimental.pallas import tpu as pltpu
from jax.experimental.pallas import tpu_sc as plsc

EMBED = 768
B, S = 4, 2048
NROWS = B * S

NC, NS = 2, 16
NW = NC * NS
SBLK = S // NW
CHS = 16
NSC = SBLK // CHS
ROWS = B * CHS
LANES = 16
VECS = EMBED // LANES
GROUP = 24


def _emb_body(ids_hbm, wte_hbm, wpe_hbm, out_hbm,
              idx_v, idx2, g0, g1, p0, p1,
              gs0, gs1, ps0, ps1, os0, os1, isem):
    wid = lax.axis_index("s") * NC + lax.axis_index("c")
    s_base = wid * SBLK

    idx_cp = [pltpu.async_copy(ids_hbm.at[b, wid],
                               idx_v.at[pl.ds(b * NSC, NSC)], isem)
              for b in range(B)]
    for cp in idx_cp:
        cp.wait()
    for b in range(B):
        for x in range(NSC):
            idx2[x, pl.ds(b * CHS, CHS)] = idx_v[b * NSC + x, :]

    gbuf = (g0, g1)
    pbuf = (p0, p1)
    gsem = (gs0, gs1)
    psem = (ps0, ps1)
    osem = (os0, os1)

    def gissue(x):
        bg = x % 2
        return [pltpu.async_copy(wte_hbm.at[idx2.at[x]], gbuf[bg],
                                 gsem[bg])]

    def pissue(x):
        bp = x % 2
        return pltpu.async_copy(
            wpe_hbm.at[pl.ds(s_base + x * CHS, CHS)], pbuf[bp], psem[bp])

    def add_super(bg, bp):
        def row_body(r, carry):
            for j0 in range(0, VECS, GROUP):
                xs = [pbuf[bp][r, pl.ds((j0 + k) * LANES, LANES)]
                      for k in range(GROUP)]
                for b in range(B):
                    for k in range(GROUP):
                        plsc.addupdate(
                            gbuf[bg].at[b * CHS + r,
                                        pl.ds((j0 + k) * LANES, LANES)],
                            xs[k])
            return carry
        lax.fori_loop(0, CHS, row_body, 0)

    def oissue(x):
        bg = x % 2
        return [pltpu.async_copy(gbuf[bg].at[pl.ds(b * CHS, CHS)],
                                 out_hbm.at[pl.ds(b * S + s_base + x * CHS,
                                                  CHS)],
                                 osem[bg])
                for b in range(B)]

    pending_g = {0: gissue(0)}
    pending_p = {0: pissue(0)}
    out_cp = {}
    for x in range(NSC):
        bg, bp = x % 2, x % 2
        if x + 1 < NSC:
            if x >= 1:
                for cp in out_cp.pop(x - 1):
                    cp.wait()
            pending_g[x + 1] = gissue(x + 1)
            pending_p[x + 1] = pissue(x + 1)
        for cp in pending_g.pop(x):
            cp.wait()
        pending_p.pop(x).wait()
        add_super(bg, bp)
        out_cp[x] = oissue(x)
    for x in sorted(out_cp):
        for cp in out_cp.pop(x):
            cp.wait()


@functools.partial(
    pl.kernel,
    mesh=plsc.VectorSubcoreMesh(core_axis_name="c", subcore_axis_name="s"),
    out_type=jax.ShapeDtypeStruct((NROWS, EMBED), jnp.float32),
    scratch_types=[
        pltpu.VMEM((B * NSC, CHS), jnp.int32),
        pltpu.VMEM((NSC, ROWS), jnp.int32),
        pltpu.VMEM((ROWS, EMBED), jnp.float32),
        pltpu.VMEM((ROWS, EMBED), jnp.float32),
        pltpu.VMEM((CHS, EMBED), jnp.float32),
        pltpu.VMEM((CHS, EMBED), jnp.float32),
        pltpu.SemaphoreType.DMA,
        pltpu.SemaphoreType.DMA,
        pltpu.SemaphoreType.DMA,
        pltpu.SemaphoreType.DMA,
        pltpu.SemaphoreType.DMA,
        pltpu.SemaphoreType.DMA,
        pltpu.SemaphoreType.DMA,
    ],
)
def _emb(ids_hbm, wte_hbm, wpe_hbm, out_hbm, *scratch):
    _emb_body(ids_hbm, wte_hbm, wpe_hbm, out_hbm, *scratch)


def kernel(input_ids, wte, wpe):
    batch, seq = input_ids.shape
    ids4 = input_ids.astype(jnp.int32).reshape(batch, NW, NSC, CHS)
    out = _emb(ids4, wte, wpe)
    return out.reshape(batch, seq, EMBED)


# speedup vs baseline: 1.0020x; 1.0020x over previous
import functools

import jax
import jax.numpy as jnp
from jax import lax
from jax.experimental import pallas as pl
from jax.experimental.pallas import tpu as pltpu
from jax.experimental.pallas import tpu_sc as plsc

EMBED = 768
B, S = 4, 2048
NROWS = B * S

NC, NS = 2, 16
NW = NC * NS
SBLK = S // NW
CHS = 16
NSC = SBLK // CHS
ROWS = B * CHS
LANES = 16
VECS = EMBED // LANES
GROUP = 48


def _emb_body(ids_hbm, wte_hbm, wpe_hbm, out_hbm,
              idx_v, idx2, g0, g1, p0, p1,
              gs0, gs1, ps0, ps1, os0, os1, isem):
    wid = lax.axis_index("s") * NC + lax.axis_index("c")
    s_base = wid * SBLK

    idx_cp = [pltpu.async_copy(ids_hbm.at[b, wid],
                               idx_v.at[pl.ds(b * NSC, NSC)], isem)
              for b in range(B)]
    for cp in idx_cp:
        cp.wait()
    for b in range(B):
        for x in range(NSC):
            idx2[x, pl.ds(b * CHS, CHS)] = idx_v[b * NSC + x, :]

    gbuf = (g0, g1)
    pbuf = (p0, p1)
    gsem = (gs0, gs1)
    psem = (ps0, ps1)
    osem = (os0, os1)

    def gissue(x):
        bg = x % 2
        return [pltpu.async_copy(wte_hbm.at[idx2.at[x]], gbuf[bg],
                                 gsem[bg])]

    def pissue(x):
        bp = x % 2
        return pltpu.async_copy(
            wpe_hbm.at[pl.ds(s_base + x * CHS, CHS)], pbuf[bp], psem[bp])

    def add_super(bg, bp):
        def row_body(r, carry):
            for j0 in range(0, VECS, GROUP):
                xs = [pbuf[bp][r, pl.ds((j0 + k) * LANES, LANES)]
                      for k in range(GROUP)]
                for b in range(B):
                    for k in range(GROUP):
                        plsc.addupdate(
                            gbuf[bg].at[b * CHS + r,
                                        pl.ds((j0 + k) * LANES, LANES)],
                            xs[k])
            return carry
        lax.fori_loop(0, CHS, row_body, 0)

    def oissue(x):
        bg = x % 2
        return [pltpu.async_copy(gbuf[bg].at[pl.ds(b * CHS, CHS)],
                                 out_hbm.at[pl.ds(b * S + s_base + x * CHS,
                                                  CHS)],
                                 osem[bg])
                for b in range(B)]

    pending_g = {0: gissue(0)}
    pending_p = {0: pissue(0)}
    out_cp = {}
    for x in range(NSC):
        bg, bp = x % 2, x % 2
        if x + 1 < NSC:
            if x >= 1:
                for cp in out_cp.pop(x - 1):
                    cp.wait()
            pending_g[x + 1] = gissue(x + 1)
            pending_p[x + 1] = pissue(x + 1)
        for cp in pending_g.pop(x):
            cp.wait()
        pending_p.pop(x).wait()
        add_super(bg, bp)
        out_cp[x] = oissue(x)
    for x in sorted(out_cp):
        for cp in out_cp.pop(x):
            cp.wait()


@functools.partial(
    pl.kernel,
    mesh=plsc.VectorSubcoreMesh(core_axis_name="c", subcore_axis_name="s"),
    out_type=jax.ShapeDtypeStruct((NROWS, EMBED), jnp.float32),
    scratch_types=[
        pltpu.VMEM((B * NSC, CHS), jnp.int32),
        pltpu.VMEM((NSC, ROWS), jnp.int32),
        pltpu.VMEM((ROWS, EMBED), jnp.float32),
        pltpu.VMEM((ROWS, EMBED), jnp.float32),
        pltpu.VMEM((CHS, EMBED), jnp.float32),
        pltpu.VMEM((CHS, EMBED), jnp.float32),
        pltpu.SemaphoreType.DMA,
        pltpu.SemaphoreType.DMA,
        pltpu.SemaphoreType.DMA,
        pltpu.SemaphoreType.DMA,
        pltpu.SemaphoreType.DMA,
        pltpu.SemaphoreType.DMA,
        pltpu.SemaphoreType.DMA,
    ],
)
def _emb(ids_hbm, wte_hbm, wpe_hbm, out_hbm, *scratch):
    _emb_body(ids_hbm, wte_hbm, wpe_hbm, out_hbm, *scratch)


def kernel(input_ids, wte, wpe):
    batch, seq = input_ids.shape
    ids4 = input_ids.astype(jnp.int32).reshape(batch, NW, NSC, CHS)
    out = _emb(ids4, wte, wpe)
    return out.reshape(batch, seq, EMBED)
